# Initial kernel scaffold; baseline (speedup 1.0000x reference)
import jax
import jax.numpy as jnp
from jax.experimental import pallas as pl


def kernel(_input, dependency_triples, W, a):
    raise NotImplementedError("write your pallas kernel here")


# speedup vs baseline: 21.5481x; 21.5481x over previous
import functools

import jax
import jax.numpy as jnp
from jax import lax
from jax.experimental import pallas as pl
from jax.experimental.pallas import tpu as pltpu
from jax.experimental.pallas import tpu_sc as plsc

ALPHA = 0.2


def _tc_matmul_body(x_ref, wt_ref, a12_ref, wx_ref, s12_ref):
    wx = jnp.dot(x_ref[...], wt_ref[...], preferred_element_type=jnp.float32)
    wx_ref[...] = wx
    s12_ref[...] = jnp.dot(wx, a12_ref[...], preferred_element_type=jnp.float32)


def kernel(_input, dependency_triples, W, a):
    x = _input
    N, D = x.shape
    L = 16
    NC, NS = 2, 16
    NW = NC * NS
    CHUNK = ((N + NW - 1) // NW + L - 1) // L * L
    NPAD = NW * CHUNK
    SUB = 80
    NSUB = CHUNK // SUB

    wt = W.T
    a12 = a.reshape(2, D).T
    BLK = 1000
    wx, s12 = pl.pallas_call(
        _tc_matmul_body,
        grid=(N // BLK,),
        in_specs=[
            pl.BlockSpec((BLK, D), lambda i: (i, 0)),
            pl.BlockSpec((D, D), lambda i: (0, 0)),
            pl.BlockSpec((D, 2), lambda i: (0, 0)),
        ],
        out_specs=[
            pl.BlockSpec((BLK, D), lambda i: (i, 0)),
            pl.BlockSpec((BLK, 2), lambda i: (i, 0)),
        ],
        out_shape=[
            jax.ShapeDtypeStruct((N, D), jnp.float32),
            jax.ShapeDtypeStruct((N, 2), jnp.float32),
        ],
    )(x, wt, a12)

    mesh = plsc.VectorSubcoreMesh(core_axis_name="c", subcore_axis_name="s")

    @functools.partial(
        pl.kernel,
        mesh=mesh,
        compiler_params=pltpu.CompilerParams(needs_layout_passes=False),
        out_type=jax.ShapeDtypeStruct((NPAD, D), jnp.float32),
        scratch_types=[
            pltpu.VMEM((3 * N,), jnp.int32),
            pltpu.VMEM((2 * N,), jnp.float32),
            pltpu.VMEM((CHUNK,), jnp.int32),
            pltpu.VMEM((CHUNK,), jnp.float32),
            pltpu.VMEM((NSUB, SUB), jnp.int32),
            pltpu.VMEM((NSUB, SUB), jnp.int32),
            pltpu.VMEM((SUB, D), jnp.float32),
            pltpu.VMEM((SUB, D), jnp.float32),
            pltpu.SemaphoreType.DMA,
        ],
    )
    def _sc_gat(dt_hbm, wx_hbm, s12_hbm, out_hbm,
                dt_v, s12_v, inv_v, coef_v, pidx_v, qidx_v, prow_v, qrow_v,
                sem):
        cid = lax.axis_index("c")
        sid = lax.axis_index("s")
        wid = sid * NC + cid
        base = wid * CHUNK

        pltpu.sync_copy(dt_hbm, dt_v)
        pltpu.sync_copy(s12_hbm, s12_v)

        lanes = lax.iota(jnp.int32, L)

        def _scan_body(k, carry):
            ivec = k * L + lanes
            g = plsc.load_gather(dt_v, [ivec * 3 + 2])
            m = (g >= base) & (g < base + CHUNK)
            idx = jnp.where(m, g - base, 0)
            plsc.store_scatter(inv_v, [idx], ivec, mask=m)
            return carry

        lax.fori_loop(0, N // L, _scan_body, 0)

        for t in range(CHUNK // L):
            jvec = base + t * L + lanes
            jc = jnp.minimum(jvec, N - 1)
            g = plsc.load_gather(dt_v, [jc * 3 + 2])
            gp = jnp.clip(g, 0, N - 1)
            q = inv_v[pl.ds(t * L, L)]
            qp = jnp.clip(q, 0, N - 1)
            s1v = plsc.load_gather(s12_v, [jc * 2])
            s2g = plsc.load_gather(s12_v, [qp * 2 + 1])
            e = s1v + s2g
            coef = jnp.where(e > 0, jnp.float32(1.0), jnp.float32(1.0 / N))
            coef_v[pl.ds(t * L, L)] = coef
            sci, off = t // (SUB // L), (t % (SUB // L)) * L
            pidx_v[sci, pl.ds(off, L)] = gp
            qidx_v[sci, pl.ds(off, L)] = qp

        for sci in range(NSUB):
            pltpu.async_copy(wx_hbm.at[pidx_v.at[sci]], prow_v, sem).wait()
            pltpu.async_copy(wx_hbm.at[qidx_v.at[sci]], qrow_v, sem).wait()

            def _row_body(r, carry):
                cb = plsc.load_gather(
                    coef_v, [jnp.full((L,), sci * SUB, jnp.int32) + r])
                for cix in range(D // L):
                    p = prow_v[r, pl.ds(cix * L, L)]
                    qv = qrow_v[r, pl.ds(cix * L, L)]
                    h = p + cb * qv
                    o = jnp.where(h > 0, h, jnp.float32(ALPHA) * h)
                    prow_v[r, pl.ds(cix * L, L)] = o
                return carry

            lax.fori_loop(0, SUB, _row_body, 0)
            pltpu.sync_copy(prow_v, out_hbm.at[pl.ds(base + sci * SUB, SUB)])

    out_pad = _sc_gat(dependency_triples.reshape(-1), wx, s12.reshape(-1))
    return out_pad[:N]


# speedup vs baseline: 29.6726x; 1.3770x over previous
import functools

import jax
import jax.numpy as jnp
from jax import lax
from jax.experimental import pallas as pl
from jax.experimental.pallas import tpu as pltpu
from jax.experimental.pallas import tpu_sc as plsc

ALPHA = 0.2


def _tc_matmul_body(x_ref, wt_ref, a12_ref, wx_ref, s12_ref):
    wx = jnp.dot(x_ref[...], wt_ref[...], preferred_element_type=jnp.float32)
    wx_ref[...] = wx
    s12_ref[...] = jnp.dot(wx, a12_ref[...], preferred_element_type=jnp.float32)


def kernel(_input, dependency_triples, W, a):
    x = _input
    N, D = x.shape
    L = 16
    NC, NS = 2, 16
    NW = NC * NS
    CHUNK = ((N + NW - 1) // NW + L - 1) // L * L
    SUB = 80
    NSUB = CHUNK // SUB
    NOUT = N + 8

    wt = W.T
    a12 = a.reshape(2, D).T
    BLK = 1000
    wx, s12 = pl.pallas_call(
        _tc_matmul_body,
        grid=(N // BLK,),
        in_specs=[
            pl.BlockSpec((BLK, D), lambda i: (i, 0)),
            pl.BlockSpec((D, D), lambda i: (0, 0)),
            pl.BlockSpec((D, 2), lambda i: (0, 0)),
        ],
        out_specs=[
            pl.BlockSpec((BLK, D), lambda i: (i, 0)),
            pl.BlockSpec((BLK, 2), lambda i: (i, 0)),
        ],
        out_shape=[
            jax.ShapeDtypeStruct((N, D), jnp.float32),
            jax.ShapeDtypeStruct((N, 2), jnp.float32),
        ],
    )(x, wt, a12)

    mesh = plsc.VectorSubcoreMesh(core_axis_name="c", subcore_axis_name="s")

    @functools.partial(
        pl.kernel,
        mesh=mesh,
        compiler_params=pltpu.CompilerParams(needs_layout_passes=False),
        out_type=jax.ShapeDtypeStruct((NOUT, D), jnp.float32),
        scratch_types=[
            pltpu.VMEM((N,), jnp.int32),
            pltpu.VMEM((N,), jnp.float32),
            pltpu.VMEM((CHUNK,), jnp.float32),
            pltpu.VMEM((CHUNK,), jnp.float32),
            pltpu.VMEM((NSUB, SUB), jnp.int32),
            pltpu.VMEM((NSUB, SUB), jnp.int32),
            pltpu.VMEM((2, SUB, D), jnp.float32),
            pltpu.VMEM((2, SUB, D), jnp.float32),
            pltpu.SemaphoreType.DMA,
            pltpu.SemaphoreType.DMA,
            pltpu.SemaphoreType.DMA,
        ],
    )
    def _sc_gat(gov_hbm, wx_hbm, s1_hbm, s2_hbm, out_hbm,
                gov_v, s1_v, s2c_v, coef_v, pidx_v, didx_v, prow_v, qrow_v,
                semp, semq, semo):
        cid = lax.axis_index("c")
        sid = lax.axis_index("s")
        wid = sid * NC + cid
        base = wid * CHUNK

        pltpu.sync_copy(gov_hbm, gov_v)
        pltpu.sync_copy(s1_hbm, s1_v)
        for sci in range(NSUB):
            qbase = jnp.minimum(base + sci * SUB, N - SUB)
            pltpu.sync_copy(s2_hbm.at[pl.ds(qbase, SUB)],
                            s2c_v.at[pl.ds(sci * SUB, SUB)])

        lanes = lax.iota(jnp.int32, L)

        for t in range(CHUNK // L):
            ivec = base + t * L + lanes
            ic = jnp.minimum(ivec, N - 1)
            g = plsc.load_gather(gov_v, [ic])
            gp = jnp.clip(g, 0, N - 1)
            gg = plsc.load_gather(gov_v, [gp])
            s1g = plsc.load_gather(s1_v, [gp])
            s2i = s2c_v[pl.ds(t * L, L)]
            e = s1g + s2i
            coef = jnp.where(e > 0, jnp.float32(1.0), jnp.float32(1.0 / N))
            coef_v[pl.ds(t * L, L)] = coef
            dest = jnp.clip(jnp.where(ivec < N, g, N), 0, N)
            sci, off = t // (SUB // L), (t % (SUB // L)) * L
            pidx_v[sci, pl.ds(off, L)] = jnp.clip(gg, 0, N - 1)
            didx_v[sci, pl.ds(off, L)] = dest

        def _start(sci):
            buf = sci % 2
            qbase = jnp.minimum(base + sci * SUB, N - SUB)
            hp = pltpu.async_copy(wx_hbm.at[pidx_v.at[sci]], prow_v.at[buf],
                                  semp)
            hq = pltpu.async_copy(wx_hbm.at[pl.ds(qbase, SUB)],
                                  qrow_v.at[buf], semq)
            return hp, hq

        inflight = _start(0)
        scat = None
        for sci in range(NSUB):
            buf = sci % 2
            hp, hq = inflight
            hp.wait()
            hq.wait()
            if scat is not None:
                scat.wait()
                scat = None
            if sci + 1 < NSUB:
                inflight = _start(sci + 1)

            def _row_body(r, carry, _sci=sci, _buf=buf):
                cb = plsc.load_gather(
                    coef_v, [jnp.full((L,), _sci * SUB, jnp.int32) + r])
                for cix in range(D // L):
                    p = prow_v[_buf, r, pl.ds(cix * L, L)]
                    qv = qrow_v[_buf, r, pl.ds(cix * L, L)]
                    h = p + cb * qv
                    o = jnp.maximum(h, jnp.float32(ALPHA) * h)
                    prow_v[_buf, r, pl.ds(cix * L, L)] = o
                return carry

            lax.fori_loop(0, SUB, _row_body, 0, unroll=4)
            scat = pltpu.async_copy(prow_v.at[buf], out_hbm.at[didx_v.at[sci]],
                                    semo)
        scat.wait()

    gov = dependency_triples[:, 2]
    out_pad = _sc_gat(gov, wx, s12[:, 0], s12[:, 1])
    return out_pad[:N]


# speedup vs baseline: 39.3846x; 1.3273x over previous
import functools

import jax
import jax.numpy as jnp
from jax import lax
from jax.experimental import pallas as pl
from jax.experimental.pallas import tpu as pltpu
from jax.experimental.pallas import tpu_sc as plsc

ALPHA = 0.2


def _tc_matmul_body(x_ref, wt_ref, a1_ref, a2_ref, wx_ref, s1_ref, s2_ref):
    wx = jnp.dot(x_ref[...], wt_ref[...], preferred_element_type=jnp.float32)
    wx_ref[...] = wx
    s1_ref[...] = jnp.dot(wx, a1_ref[...], preferred_element_type=jnp.float32)
    s2_ref[...] = jnp.dot(wx, a2_ref[...], preferred_element_type=jnp.float32)


def kernel(_input, dependency_triples, W, a):
    x = _input
    N, D = x.shape
    L = 16
    NC, NS = 2, 16
    NW = NC * NS
    CHUNK = ((N + NW - 1) // NW + L - 1) // L * L
    SUB = 80
    NSUB = CHUNK // SUB

    wt = W.T
    a1 = a[:, :D].T
    a2 = a[:, D:].T
    BLK = 1000
    wx, s1, s2 = pl.pallas_call(
        _tc_matmul_body,
        grid=(N // BLK,),
        in_specs=[
            pl.BlockSpec((BLK, D), lambda i: (i, 0)),
            pl.BlockSpec((D, D), lambda i: (0, 0)),
            pl.BlockSpec((D, 1), lambda i: (0, 0)),
            pl.BlockSpec((D, 1), lambda i: (0, 0)),
        ],
        out_specs=[
            pl.BlockSpec((BLK, D), lambda i: (i, 0)),
            pl.BlockSpec((BLK, 1), lambda i: (i, 0)),
            pl.BlockSpec((BLK, 1), lambda i: (i, 0)),
        ],
        out_shape=[
            jax.ShapeDtypeStruct((N, D), jnp.float32),
            jax.ShapeDtypeStruct((N, 1), jnp.float32),
            jax.ShapeDtypeStruct((N, 1), jnp.float32),
        ],
    )(x, wt, a1, a2)

    mesh = plsc.VectorSubcoreMesh(core_axis_name="c", subcore_axis_name="s")

    @functools.partial(
        pl.kernel,
        mesh=mesh,
        compiler_params=pltpu.CompilerParams(needs_layout_passes=False),
        out_type=jax.ShapeDtypeStruct((N, D), jnp.float32),
        scratch_types=[
            pltpu.VMEM((N,), jnp.int32),
            pltpu.VMEM((N,), jnp.float32),
            pltpu.VMEM((CHUNK,), jnp.float32),
            pltpu.VMEM((CHUNK,), jnp.float32),
            pltpu.VMEM((CHUNK,), jnp.int32),
            pltpu.VMEM((CHUNK,), jnp.int32),
            pltpu.VMEM((NSUB, SUB), jnp.int32),
            pltpu.VMEM((NSUB, SUB), jnp.int32),
            pltpu.VMEM((2, SUB, D), jnp.float32),
            pltpu.VMEM((2, SUB, D), jnp.float32),
            pltpu.VMEM((2, SUB, D), jnp.float32),
            pltpu.SemaphoreType.DMA,
            pltpu.SemaphoreType.DMA,
            pltpu.SemaphoreType.DMA,
            pltpu.SemaphoreType.DMA,
        ],
    )
    def _sc_gat(gov_hbm, wx_hbm, s1_hbm, s2_hbm, out_hbm,
                gov_v, s1_v, s2c_v, coef_v, pidx_f, didx_f, pidx_v, didx_v,
                prow_v, qrow_v, orow_v, sems, semp, semq, semo):
        cid = lax.axis_index("c")
        sid = lax.axis_index("s")
        wid = sid * NC + cid
        base = wid * CHUNK

        hst = [pltpu.async_copy(gov_hbm, gov_v, sems),
               pltpu.async_copy(s1_hbm, s1_v, sems)]
        for sci in range(NSUB):
            qb = jnp.minimum(base + sci * SUB, N - SUB)
            hst.append(pltpu.async_copy(s2_hbm.at[pl.ds(qb, SUB)],
                                        s2c_v.at[pl.ds(sci * SUB, SUB)],
                                        sems))
        hq = {}
        for sci in range(2):
            qb = jnp.minimum(base + sci * SUB, N - SUB)
            hq[sci] = pltpu.async_copy(wx_hbm.at[pl.ds(qb, SUB)],
                                       qrow_v.at[sci % 2], semq)
        for h in hst:
            h.wait()

        lanes = lax.iota(jnp.int32, L)

        def _prep(t):
            ivec = base + t * L + lanes
            ic = jnp.where(ivec < N, ivec, ivec % SUB + (N - SUB))
            g = plsc.load_gather(gov_v, [ic])
            gg = plsc.load_gather(gov_v, [g])
            s1g = plsc.load_gather(s1_v, [g])
            s2i = s2c_v[pl.ds(t * L, L)]
            e = s1g + s2i
            coef = jnp.where(e > 0, jnp.float32(1.0), jnp.float32(1.0 / N))
            coef_v[pl.ds(t * L, L)] = coef
            pidx_f[pl.ds(t * L, L)] = gg
            didx_f[pl.ds(t * L, L)] = g

        def _xfer(sci):
            for k in range(SUB // L):
                o = sci * SUB + k * L
                pidx_v[sci, pl.ds(k * L, L)] = pidx_f[pl.ds(o, L)]
                didx_v[sci, pl.ds(k * L, L)] = didx_f[pl.ds(o, L)]

        plsc.parallel_loop(0, SUB // L)(_prep)
        _xfer(0)
        hp = {0: pltpu.async_copy(wx_hbm.at[pidx_v.at[0]], prow_v.at[0],
                                  semp)}
        plsc.parallel_loop(SUB // L, CHUNK // L)(_prep)
        for sci in range(1, NSUB):
            _xfer(sci)
        hp[1] = pltpu.async_copy(wx_hbm.at[pidx_v.at[1]], prow_v.at[1], semp)

        scat = {}
        for sci in range(NSUB):
            buf = sci % 2
            hp[sci].wait()
            hq[sci].wait()
            if sci >= 2:
                scat[sci - 2].wait()

            @plsc.parallel_loop(0, SUB, unroll=2)
            def _row(r, _sci=sci, _buf=buf):
                cb = plsc.load_gather(
                    coef_v, [jnp.full((L,), _sci * SUB, jnp.int32) + r])
                for cix in range(D // L):
                    p = prow_v[_buf, r, pl.ds(cix * L, L)]
                    qv = qrow_v[_buf, r, pl.ds(cix * L, L)]
                    h = p + cb * qv
                    orow_v[_buf, r, pl.ds(cix * L, L)] = jnp.maximum(
                        h, jnp.float32(ALPHA) * h)

            scat[sci] = pltpu.async_copy(orow_v.at[buf],
                                         out_hbm.at[didx_v.at[sci]], semo)
            if sci + 2 < NSUB:
                hp[sci + 2] = pltpu.async_copy(wx_hbm.at[pidx_v.at[sci + 2]],
                                               prow_v.at[buf], semp)
                qb = jnp.minimum(base + (sci + 2) * SUB, N - SUB)
                hq[sci + 2] = pltpu.async_copy(wx_hbm.at[pl.ds(qb, SUB)],
                                               qrow_v.at[buf], semq)
        scat[NSUB - 2].wait()
        scat[NSUB - 1].wait()

    gov = dependency_triples[:, 2]
    return _sc_gat(gov, wx, s1.reshape(N), s2.reshape(N))
